# baseline (speedup 1.0000x reference)
import jax
import jax.numpy as jnp
from jax.experimental import pallas as pl
from jax.experimental.pallas import tpu as pltpu

B, N, M, D, H, K = 4, 1024, 1024, 512, 8, 32
DK = D // H
SCALE = DK ** -0.5
NBLK = 256
NGRID = N // NBLK

F32 = jnp.float32
I16 = jnp.int16
I32 = jnp.int32
BF16 = jnp.bfloat16


def _prefix_sum_lanes(x):
    rows, n = x.shape
    s = 1
    while s < n:
        shifted = jnp.concatenate(
            [jnp.zeros((rows, s), x.dtype), x[:, : n - s]], axis=1)
        x = x + shifted
        s *= 2
    return x


def _topk_mask(dist):
    rows = dist.shape[0]
    u = jax.lax.bitcast_convert_type(dist, I32)

    ones_col = jnp.ones((M, 1), BF16)
    one_b = jnp.bfloat16(1.0)
    zero_b = jnp.bfloat16(0.0)

    def _count(cmpb):
        sel = jnp.where(cmpb, one_b, zero_b)
        return jax.lax.dot_general(sel, ones_col, (((1,), (0,)), ((), ())),
                                   preferred_element_type=F32)

    u_hi = jax.lax.shift_right_logical(u, 16).astype(I16)
    lo1 = jnp.zeros((rows, 1), I32)
    hi1 = jnp.full((rows, 1), 32767, I32)

    def body1(_, carry):
        lo, hi = carry
        mid = lo + jax.lax.shift_right_logical(hi - lo, 1)
        ge = _count(u_hi <= mid.astype(I16)) >= K
        hi = jnp.where(ge, mid, hi)
        lo = jnp.where(ge, lo, mid + 1)
        return lo, hi

    lo1, hi1 = jax.lax.fori_loop(0, 15, body1, (lo1, hi1))
    tau_hi = lo1.astype(I16)

    cnt_lt_hi = _count(u_hi < tau_hi)
    k2 = K - cnt_lt_hi

    eq_hi = u_hi == tau_hi
    lo16s = ((u & 0xFFFF) - 32768).astype(I16)
    v = jnp.where(eq_hi, lo16s, jnp.int16(32767))

    lo2 = jnp.full((rows, 1), -32768, I32)
    hi2 = jnp.full((rows, 1), 32767, I32)

    def body2(_, carry):
        lo, hi = carry
        mid = lo + jax.lax.shift_right_logical(hi - lo, 1)
        ge = _count(v <= mid.astype(I16)) >= k2
        hi = jnp.where(ge, mid, hi)
        lo = jnp.where(ge, lo, mid + 1)
        return lo, hi

    lo2, hi2 = jax.lax.fori_loop(0, 16, body2, (lo2, hi2))
    tau = (jax.lax.shift_left(lo1, 16) | ((lo2 + 32768) & 0xFFFF))

    lt = u < tau
    eq = u == tau
    cnt_lt = cnt_lt_hi + _count(v < lo2.astype(I16))
    extra = (K - cnt_lt).astype(I16)
    rank = _prefix_sum_lanes(eq.astype(I16))
    tie_sel = eq & (rank <= extra)
    return lt | tie_sel


def _kv_body(tgt_fea_ref, wk_ref, bk_ref, wv_ref, bv_ref, km_ref, vm_ref):
    tf = tgt_fea_ref[0]
    km = jax.lax.dot_general(tf, wk_ref[...], (((1,), (0,)), ((), ())),
                             preferred_element_type=F32) + bk_ref[...]
    vm = jax.lax.dot_general(tf, wv_ref[...], (((1,), (0,)), ((), ())),
                             preferred_element_type=F32) + bv_ref[...]
    km_ref[0] = km
    vm_ref[0] = vm


def _attn_body(src_ref, tgt_ref, src_fea_ref, km_ref, vm_ref,
               wq_ref, bq_ref, w1_ref, b1_ref, gamma_ref, beta_ref,
               w2_ref, b2_ref, updated_ref, avg_attn_ref):
    src = src_ref[0]
    tgt = tgt_ref[0]
    src_fea = src_fea_ref[0]
    km = km_ref[0]
    vm = vm_ref[0]

    st = jax.lax.dot_general(src, tgt, (((1,), (1,)), ((), ())),
                             preferred_element_type=F32)
    s2 = jnp.sum(src * src, axis=1, keepdims=True)
    t2 = jnp.sum(tgt * tgt, axis=1, keepdims=True)
    d2 = s2 + t2.reshape(1, M) - 2.0 * st
    dist = jnp.sqrt(jnp.maximum(d2, 0.0))

    mask = _topk_mask(dist)

    q = jax.lax.dot_general(src_fea, wq_ref[...],
                            (((1,), (0,)), ((), ())),
                            preferred_element_type=F32) + bq_ref[...]
    q = q * SCALE

    neg_inf = jnp.float32(-jnp.inf)
    avg = jnp.zeros((NBLK, M), F32)
    outs = []
    for h in range(H):
        sl = slice(h * DK, (h + 1) * DK)
        s = jax.lax.dot_general(q[:, sl], km[:, sl], (((1,), (1,)), ((), ())),
                                preferred_element_type=F32)
        s = jnp.where(mask, s, neg_inf)
        mx = jnp.max(s, axis=1, keepdims=True)
        p = jnp.exp(s - mx)
        denom = jnp.sum(p, axis=1, keepdims=True)
        a = p / denom
        avg = avg + a
        outs.append(jax.lax.dot_general(a, vm[:, sl],
                                        (((1,), (0,)), ((), ())),
                                        preferred_element_type=F32))
    avg_attn_ref[0] = avg * (1.0 / H)
    out = jnp.concatenate(outs, axis=1)

    hh = out + src_fea
    l1 = jax.lax.dot_general(hh, w1_ref[...],
                             (((1,), (0,)), ((), ())),
                             preferred_element_type=F32) + b1_ref[...]
    mu = jnp.mean(l1, axis=-1, keepdims=True)
    var = jnp.mean((l1 - mu) ** 2, axis=-1, keepdims=True)
    ln = gamma_ref[...] * (l1 - mu) / jnp.sqrt(var + 1e-5) + beta_ref[...]
    act = jnp.maximum(ln, 0.0)
    updated_ref[0] = jax.lax.dot_general(
        act, w2_ref[...], (((1,), (0,)), ((), ())),
        preferred_element_type=F32) + b2_ref[...]


@jax.jit
def kernel(src, tgt, src_fea, tgt_fea, Wq, bq, Wk, bk, Wv, bv,
           W1, b1, gamma, beta, W2, b2):
    vecs = [v.reshape(1, D) for v in (bq, bk, bv, b1, gamma, beta, b2)]
    bq2, bk2, bv2, b12, gamma2, beta2, b22 = vecs

    wspec = pl.BlockSpec((D, D), lambda *_: (0, 0))
    vspec = pl.BlockSpec((1, D), lambda *_: (0, 0))

    km, vm = pl.pallas_call(
        _kv_body,
        grid=(B,),
        in_specs=[
            pl.BlockSpec((1, M, D), lambda b: (b, 0, 0)),
            wspec, vspec, wspec, vspec,
        ],
        out_specs=[pl.BlockSpec((1, M, D), lambda b: (b, 0, 0))] * 2,
        out_shape=[jax.ShapeDtypeStruct((B, M, D), F32)] * 2,
        compiler_params=pltpu.CompilerParams(
            dimension_semantics=("arbitrary",),
        ),
    )(tgt_fea, Wk, bk2, Wv, bv2)

    updated, avg_attn = pl.pallas_call(
        _attn_body,
        grid=(B, NGRID),
        in_specs=[
            pl.BlockSpec((1, NBLK, 3), lambda b, n: (b, n, 0)),
            pl.BlockSpec((1, M, 3), lambda b, n: (b, 0, 0)),
            pl.BlockSpec((1, NBLK, D), lambda b, n: (b, n, 0)),
            pl.BlockSpec((1, M, D), lambda b, n: (b, 0, 0)),
            pl.BlockSpec((1, M, D), lambda b, n: (b, 0, 0)),
            wspec, vspec, wspec, vspec, vspec, vspec, wspec, vspec,
        ],
        out_specs=[
            pl.BlockSpec((1, NBLK, D), lambda b, n: (b, n, 0)),
            pl.BlockSpec((1, NBLK, M), lambda b, n: (b, n, 0)),
        ],
        out_shape=[jax.ShapeDtypeStruct((B, N, D), F32),
                   jax.ShapeDtypeStruct((B, N, M), F32)],
        compiler_params=pltpu.CompilerParams(
            dimension_semantics=("arbitrary", "arbitrary"),
        ),
    )(src, tgt, src_fea, km, vm, Wq, bq2, W1, b12, gamma2, beta2,
      W2, b22)
    return updated, avg_attn


# speedup vs baseline: 1.3457x; 1.3457x over previous
import jax
import jax.numpy as jnp
from jax.experimental import pallas as pl
from jax.experimental.pallas import tpu as pltpu

B, N, M, D, H, K = 4, 1024, 1024, 512, 8, 32
DK = D // H
SCALE = DK ** -0.5
NBLK = 512
NGRID = N // NBLK

F32 = jnp.float32
I16 = jnp.int16
I32 = jnp.int32
BF16 = jnp.bfloat16


def _prefix_sum_lanes(x):
    rows, n = x.shape
    s = 1
    while s < n:
        shifted = jnp.concatenate(
            [jnp.zeros((rows, s), x.dtype), x[:, : n - s]], axis=1)
        x = x + shifted
        s *= 2
    return x


def _topk_mask(dist):
    u = jax.lax.bitcast_convert_type(dist, I32)

    lo = jnp.min(u, axis=1, keepdims=True)
    hi = jnp.max(u, axis=1, keepdims=True)

    def body(_, carry):
        lo, hi = carry
        mid = lo + jax.lax.shift_right_logical(hi - lo, 1)
        cnt = jnp.sum((u <= mid).astype(I32), axis=1, keepdims=True)
        ge = cnt >= K
        hi = jnp.where(ge, mid, hi)
        lo = jnp.where(ge, lo, mid + 1)
        return lo, hi

    lo, hi = jax.lax.fori_loop(0, 31, body, (lo, hi))
    tau = lo

    lt = u < tau
    eq = u == tau
    cnt_lt = jnp.sum(lt.astype(I32), axis=1, keepdims=True)
    extra = (K - cnt_lt).astype(I16)
    rank = _prefix_sum_lanes(eq.astype(I16))
    tie_sel = eq & (rank <= extra)
    return lt | tie_sel


def _kv_body(tgt_fea_ref, wk_ref, bk_ref, wv_ref, bv_ref, km_ref, vm_ref):
    tf = tgt_fea_ref[0]
    km = jax.lax.dot_general(tf, wk_ref[...], (((1,), (0,)), ((), ())),
                             preferred_element_type=F32) + bk_ref[...]
    vm = jax.lax.dot_general(tf, wv_ref[...], (((1,), (0,)), ((), ())),
                             preferred_element_type=F32) + bv_ref[...]
    km_ref[0] = km
    vm_ref[0] = vm


def _attn_body(src_ref, tgt_ref, src_fea_ref, km_ref, vm_ref,
               wq_ref, bq_ref, w1_ref, b1_ref, gamma_ref, beta_ref,
               w2_ref, b2_ref, updated_ref, avg_attn_ref):
    src = src_ref[0]
    tgt = tgt_ref[0]
    src_fea = src_fea_ref[0]
    km = km_ref[0]
    vm = vm_ref[0]

    st = jax.lax.dot_general(src, tgt, (((1,), (1,)), ((), ())),
                             preferred_element_type=F32)
    s2 = jnp.sum(src * src, axis=1, keepdims=True)
    t2 = jnp.sum(tgt * tgt, axis=1, keepdims=True)
    d2 = s2 + t2.reshape(1, M) - 2.0 * st
    dist = jnp.sqrt(jnp.maximum(d2, 0.0))

    mask = _topk_mask(dist)

    q = jax.lax.dot_general(src_fea, wq_ref[...],
                            (((1,), (0,)), ((), ())),
                            preferred_element_type=F32) + bq_ref[...]
    q = q * SCALE

    neg_inf = jnp.float32(-jnp.inf)
    avg = jnp.zeros((NBLK, M), F32)
    outs = []
    for h in range(H):
        sl = slice(h * DK, (h + 1) * DK)
        s = jax.lax.dot_general(q[:, sl], km[:, sl], (((1,), (1,)), ((), ())),
                                preferred_element_type=F32)
        s = jnp.where(mask, s, neg_inf)
        mx = jnp.max(s, axis=1, keepdims=True)
        p = jnp.exp(s - mx)
        denom = jnp.sum(p, axis=1, keepdims=True)
        a = p / denom
        avg = avg + a
        outs.append(jax.lax.dot_general(a, vm[:, sl],
                                        (((1,), (0,)), ((), ())),
                                        preferred_element_type=F32))
    avg_attn_ref[0] = avg * (1.0 / H)
    out = jnp.concatenate(outs, axis=1)

    hh = out + src_fea
    l1 = jax.lax.dot_general(hh, w1_ref[...],
                             (((1,), (0,)), ((), ())),
                             preferred_element_type=F32) + b1_ref[...]
    mu = jnp.mean(l1, axis=-1, keepdims=True)
    var = jnp.mean((l1 - mu) ** 2, axis=-1, keepdims=True)
    ln = gamma_ref[...] * (l1 - mu) / jnp.sqrt(var + 1e-5) + beta_ref[...]
    act = jnp.maximum(ln, 0.0)
    updated_ref[0] = jax.lax.dot_general(
        act, w2_ref[...], (((1,), (0,)), ((), ())),
        preferred_element_type=F32) + b2_ref[...]


@jax.jit
def kernel(src, tgt, src_fea, tgt_fea, Wq, bq, Wk, bk, Wv, bv,
           W1, b1, gamma, beta, W2, b2):
    vecs = [v.reshape(1, D) for v in (bq, bk, bv, b1, gamma, beta, b2)]
    bq2, bk2, bv2, b12, gamma2, beta2, b22 = vecs

    wspec = pl.BlockSpec((D, D), lambda *_: (0, 0))
    vspec = pl.BlockSpec((1, D), lambda *_: (0, 0))

    km, vm = pl.pallas_call(
        _kv_body,
        grid=(B,),
        in_specs=[
            pl.BlockSpec((1, M, D), lambda b: (b, 0, 0)),
            wspec, vspec, wspec, vspec,
        ],
        out_specs=[pl.BlockSpec((1, M, D), lambda b: (b, 0, 0))] * 2,
        out_shape=[jax.ShapeDtypeStruct((B, M, D), F32)] * 2,
        compiler_params=pltpu.CompilerParams(
            dimension_semantics=("arbitrary",),
        ),
    )(tgt_fea, Wk, bk2, Wv, bv2)

    updated, avg_attn = pl.pallas_call(
        _attn_body,
        grid=(B, NGRID),
        in_specs=[
            pl.BlockSpec((1, NBLK, 3), lambda b, n: (b, n, 0)),
            pl.BlockSpec((1, M, 3), lambda b, n: (b, 0, 0)),
            pl.BlockSpec((1, NBLK, D), lambda b, n: (b, n, 0)),
            pl.BlockSpec((1, M, D), lambda b, n: (b, 0, 0)),
            pl.BlockSpec((1, M, D), lambda b, n: (b, 0, 0)),
            wspec, vspec, wspec, vspec, vspec, vspec, wspec, vspec,
        ],
        out_specs=[
            pl.BlockSpec((1, NBLK, D), lambda b, n: (b, n, 0)),
            pl.BlockSpec((1, NBLK, M), lambda b, n: (b, n, 0)),
        ],
        out_shape=[jax.ShapeDtypeStruct((B, N, D), F32),
                   jax.ShapeDtypeStruct((B, N, M), F32)],
        compiler_params=pltpu.CompilerParams(
            dimension_semantics=("arbitrary", "arbitrary"),
        ),
    )(src, tgt, src_fea, km, vm, Wq, bq2, W1, b12, gamma2, beta2,
      W2, b22)
    return updated, avg_attn


# speedup vs baseline: 1.3629x; 1.0127x over previous
import jax
import jax.numpy as jnp
from jax.experimental import pallas as pl
from jax.experimental.pallas import tpu as pltpu

B, N, M, D, H, K = 4, 1024, 1024, 512, 8, 32
DK = D // H
SCALE = DK ** -0.5
NBLK = 1024
NGRID = N // NBLK

F32 = jnp.float32
I16 = jnp.int16
I32 = jnp.int32
BF16 = jnp.bfloat16


def _prefix_sum_lanes(x):
    rows, n = x.shape
    s = 1
    while s < n:
        shifted = jnp.concatenate(
            [jnp.zeros((rows, s), x.dtype), x[:, : n - s]], axis=1)
        x = x + shifted
        s *= 2
    return x


def _topk_mask(dist):
    u = jax.lax.bitcast_convert_type(dist, I32)

    lo = jnp.min(u, axis=1, keepdims=True)
    hi = jnp.max(u, axis=1, keepdims=True)

    def body(_, carry):
        lo, hi = carry
        mid = lo + jax.lax.shift_right_logical(hi - lo, 1)
        cnt = jnp.sum((u <= mid).astype(I32), axis=1, keepdims=True)
        ge = cnt >= K
        hi = jnp.where(ge, mid, hi)
        lo = jnp.where(ge, lo, mid + 1)
        return lo, hi

    lo, hi = jax.lax.fori_loop(0, 31, body, (lo, hi))
    tau = lo

    lt = u < tau
    eq = u == tau
    cnt_lt = jnp.sum(lt.astype(I32), axis=1, keepdims=True)
    extra = (K - cnt_lt).astype(I16)
    rank = _prefix_sum_lanes(eq.astype(I16))
    tie_sel = eq & (rank <= extra)
    return lt | tie_sel


def _kv_body(tgt_fea_ref, wk_ref, bk_ref, wv_ref, bv_ref, km_ref, vm_ref):
    tf = tgt_fea_ref[0]
    km = jax.lax.dot_general(tf, wk_ref[...], (((1,), (0,)), ((), ())),
                             preferred_element_type=F32) + bk_ref[...]
    vm = jax.lax.dot_general(tf, wv_ref[...], (((1,), (0,)), ((), ())),
                             preferred_element_type=F32) + bv_ref[...]
    km_ref[0] = km
    vm_ref[0] = vm


def _attn_body(src_ref, tgt_ref, src_fea_ref, km_ref, vm_ref,
               wq_ref, bq_ref, w1_ref, b1_ref, gamma_ref, beta_ref,
               w2_ref, b2_ref, updated_ref, avg_attn_ref):
    src = src_ref[0]
    tgt = tgt_ref[0]
    src_fea = src_fea_ref[0]
    km = km_ref[0]
    vm = vm_ref[0]

    st = jax.lax.dot_general(src, tgt, (((1,), (1,)), ((), ())),
                             preferred_element_type=F32)
    s2 = jnp.sum(src * src, axis=1, keepdims=True)
    t2 = jnp.sum(tgt * tgt, axis=1, keepdims=True)
    d2 = s2 + t2.reshape(1, M) - 2.0 * st
    dist = jnp.sqrt(jnp.maximum(d2, 0.0))

    mask = _topk_mask(dist)

    q = jax.lax.dot_general(src_fea, wq_ref[...],
                            (((1,), (0,)), ((), ())),
                            preferred_element_type=F32) + bq_ref[...]
    q = q * SCALE

    neg_inf = jnp.float32(-jnp.inf)
    avg = jnp.zeros((NBLK, M), F32)
    outs = []
    for h in range(H):
        sl = slice(h * DK, (h + 1) * DK)
        s = jax.lax.dot_general(q[:, sl], km[:, sl], (((1,), (1,)), ((), ())),
                                preferred_element_type=F32)
        s = jnp.where(mask, s, neg_inf)
        mx = jnp.max(s, axis=1, keepdims=True)
        p = jnp.exp(s - mx)
        denom = jnp.sum(p, axis=1, keepdims=True)
        a = p / denom
        avg = avg + a
        outs.append(jax.lax.dot_general(a, vm[:, sl],
                                        (((1,), (0,)), ((), ())),
                                        preferred_element_type=F32))
    avg_attn_ref[0] = avg * (1.0 / H)
    out = jnp.concatenate(outs, axis=1)

    hh = out + src_fea
    l1 = jax.lax.dot_general(hh, w1_ref[...],
                             (((1,), (0,)), ((), ())),
                             preferred_element_type=F32) + b1_ref[...]
    mu = jnp.mean(l1, axis=-1, keepdims=True)
    var = jnp.mean((l1 - mu) ** 2, axis=-1, keepdims=True)
    ln = gamma_ref[...] * (l1 - mu) / jnp.sqrt(var + 1e-5) + beta_ref[...]
    act = jnp.maximum(ln, 0.0)
    updated_ref[0] = jax.lax.dot_general(
        act, w2_ref[...], (((1,), (0,)), ((), ())),
        preferred_element_type=F32) + b2_ref[...]


@jax.jit
def kernel(src, tgt, src_fea, tgt_fea, Wq, bq, Wk, bk, Wv, bv,
           W1, b1, gamma, beta, W2, b2):
    vecs = [v.reshape(1, D) for v in (bq, bk, bv, b1, gamma, beta, b2)]
    bq2, bk2, bv2, b12, gamma2, beta2, b22 = vecs

    wspec = pl.BlockSpec((D, D), lambda *_: (0, 0))
    vspec = pl.BlockSpec((1, D), lambda *_: (0, 0))

    km, vm = pl.pallas_call(
        _kv_body,
        grid=(B,),
        in_specs=[
            pl.BlockSpec((1, M, D), lambda b: (b, 0, 0)),
            wspec, vspec, wspec, vspec,
        ],
        out_specs=[pl.BlockSpec((1, M, D), lambda b: (b, 0, 0))] * 2,
        out_shape=[jax.ShapeDtypeStruct((B, M, D), F32)] * 2,
        compiler_params=pltpu.CompilerParams(
            dimension_semantics=("arbitrary",),
        ),
    )(tgt_fea, Wk, bk2, Wv, bv2)

    updated, avg_attn = pl.pallas_call(
        _attn_body,
        grid=(B, NGRID),
        in_specs=[
            pl.BlockSpec((1, NBLK, 3), lambda b, n: (b, n, 0)),
            pl.BlockSpec((1, M, 3), lambda b, n: (b, 0, 0)),
            pl.BlockSpec((1, NBLK, D), lambda b, n: (b, n, 0)),
            pl.BlockSpec((1, M, D), lambda b, n: (b, 0, 0)),
            pl.BlockSpec((1, M, D), lambda b, n: (b, 0, 0)),
            wspec, vspec, wspec, vspec, vspec, vspec, wspec, vspec,
        ],
        out_specs=[
            pl.BlockSpec((1, NBLK, D), lambda b, n: (b, n, 0)),
            pl.BlockSpec((1, NBLK, M), lambda b, n: (b, n, 0)),
        ],
        out_shape=[jax.ShapeDtypeStruct((B, N, D), F32),
                   jax.ShapeDtypeStruct((B, N, M), F32)],
        compiler_params=pltpu.CompilerParams(
            dimension_semantics=("arbitrary", "arbitrary"),
            vmem_limit_bytes=80 * 1024 * 1024,
        ),
    )(src, tgt, src_fea, km, vm, Wq, bq2, W1, b12, gamma2, beta2,
      W2, b22)
    return updated, avg_attn


# speedup vs baseline: 1.4117x; 1.0358x over previous
import jax
import jax.numpy as jnp
from jax.experimental import pallas as pl
from jax.experimental.pallas import tpu as pltpu

B, N, M, D, H, K = 4, 1024, 1024, 512, 8, 32
DK = D // H
SCALE = DK ** -0.5
NBLK = 1024
NGRID = N // NBLK

F32 = jnp.float32
I16 = jnp.int16
I32 = jnp.int32
BF16 = jnp.bfloat16


def _prefix_sum_lanes(x):
    rows, n = x.shape
    s = 1
    while s < n:
        shifted = jnp.concatenate(
            [jnp.zeros((rows, s), x.dtype), x[:, : n - s]], axis=1)
        x = x + shifted
        s *= 2
    return x


def _topk_mask(dist):
    u = jax.lax.bitcast_convert_type(dist, I32)

    lo = jnp.min(u, axis=1, keepdims=True)
    hi = jnp.max(u, axis=1, keepdims=True)

    def body(_, carry):
        lo, hi = carry
        mid = lo + jax.lax.shift_right_logical(hi - lo, 1)
        cnt = jnp.sum((u <= mid).astype(I32), axis=1, keepdims=True)
        ge = cnt >= K
        hi = jnp.where(ge, mid, hi)
        lo = jnp.where(ge, lo, mid + 1)
        return lo, hi

    lo, hi = jax.lax.fori_loop(0, 31, body, (lo, hi))
    tau = lo

    lt = u < tau
    eq = u == tau
    cnt_lt = jnp.sum(lt.astype(I32), axis=1, keepdims=True)
    extra = (K - cnt_lt).astype(I16)
    rank = _prefix_sum_lanes(eq.astype(I16))
    tie_sel = eq & (rank <= extra)
    return lt | tie_sel


def _attn_body(src_ref, tgt_ref, src_fea_ref, tgt_fea_ref,
               wq_ref, bq_ref, wk_ref, bk_ref, wv_ref, bv_ref,
               w1_ref, b1_ref, gamma_ref, beta_ref,
               w2_ref, b2_ref, updated_ref, avg_attn_ref):
    src = src_ref[0]
    tgt = tgt_ref[0]
    src_fea = src_fea_ref[0]
    tgt_fea = tgt_fea_ref[0]
    km = jax.lax.dot_general(tgt_fea, wk_ref[...], (((1,), (0,)), ((), ())),
                             preferred_element_type=F32) + bk_ref[...]
    vm = jax.lax.dot_general(tgt_fea, wv_ref[...], (((1,), (0,)), ((), ())),
                             preferred_element_type=F32) + bv_ref[...]

    st = jax.lax.dot_general(src, tgt, (((1,), (1,)), ((), ())),
                             preferred_element_type=F32)
    s2 = jnp.sum(src * src, axis=1, keepdims=True)
    t2 = jnp.sum(tgt * tgt, axis=1, keepdims=True)
    d2 = s2 + t2.reshape(1, M) - 2.0 * st
    dist = jnp.sqrt(jnp.maximum(d2, 0.0))

    mask = _topk_mask(dist)

    q = jax.lax.dot_general(src_fea, wq_ref[...],
                            (((1,), (0,)), ((), ())),
                            preferred_element_type=F32) + bq_ref[...]
    q = q * SCALE

    neg_inf = jnp.float32(-jnp.inf)
    avg = jnp.zeros((NBLK, M), F32)
    outs = []
    for h in range(H):
        sl = slice(h * DK, (h + 1) * DK)
        s = jax.lax.dot_general(q[:, sl], km[:, sl], (((1,), (1,)), ((), ())),
                                preferred_element_type=F32)
        s = jnp.where(mask, s, neg_inf)
        mx = jnp.max(s, axis=1, keepdims=True)
        p = jnp.exp(s - mx)
        denom = jnp.sum(p, axis=1, keepdims=True)
        a = p / denom
        avg = avg + a
        outs.append(jax.lax.dot_general(a, vm[:, sl],
                                        (((1,), (0,)), ((), ())),
                                        preferred_element_type=F32))
    avg_attn_ref[0] = avg * (1.0 / H)
    out = jnp.concatenate(outs, axis=1)

    hh = out + src_fea
    l1 = jax.lax.dot_general(hh, w1_ref[...],
                             (((1,), (0,)), ((), ())),
                             preferred_element_type=F32) + b1_ref[...]
    mu = jnp.mean(l1, axis=-1, keepdims=True)
    var = jnp.mean((l1 - mu) ** 2, axis=-1, keepdims=True)
    ln = gamma_ref[...] * (l1 - mu) / jnp.sqrt(var + 1e-5) + beta_ref[...]
    act = jnp.maximum(ln, 0.0)
    updated_ref[0] = jax.lax.dot_general(
        act, w2_ref[...], (((1,), (0,)), ((), ())),
        preferred_element_type=F32) + b2_ref[...]


@jax.jit
def kernel(src, tgt, src_fea, tgt_fea, Wq, bq, Wk, bk, Wv, bv,
           W1, b1, gamma, beta, W2, b2):
    vecs = [v.reshape(1, D) for v in (bq, bk, bv, b1, gamma, beta, b2)]
    bq2, bk2, bv2, b12, gamma2, beta2, b22 = vecs

    wspec = pl.BlockSpec((D, D), lambda *_: (0, 0))
    vspec = pl.BlockSpec((1, D), lambda *_: (0, 0))

    updated, avg_attn = pl.pallas_call(
        _attn_body,
        grid=(B, NGRID),
        in_specs=[
            pl.BlockSpec((1, NBLK, 3), lambda b, n: (b, n, 0)),
            pl.BlockSpec((1, M, 3), lambda b, n: (b, 0, 0)),
            pl.BlockSpec((1, NBLK, D), lambda b, n: (b, n, 0)),
            pl.BlockSpec((1, M, D), lambda b, n: (b, 0, 0)),
            wspec, vspec, wspec, vspec, wspec, vspec,
            wspec, vspec, vspec, vspec, wspec, vspec,
        ],
        out_specs=[
            pl.BlockSpec((1, NBLK, D), lambda b, n: (b, n, 0)),
            pl.BlockSpec((1, NBLK, M), lambda b, n: (b, n, 0)),
        ],
        out_shape=[jax.ShapeDtypeStruct((B, N, D), F32),
                   jax.ShapeDtypeStruct((B, N, M), F32)],
        compiler_params=pltpu.CompilerParams(
            dimension_semantics=("arbitrary", "arbitrary"),
            vmem_limit_bytes=80 * 1024 * 1024,
        ),
    )(src, tgt, src_fea, tgt_fea, Wq, bq2, Wk, bk2, Wv, bv2,
      W1, b12, gamma2, beta2, W2, b22)
    return updated, avg_attn
